# Initial kernel scaffold; baseline (speedup 1.0000x reference)
import jax
import jax.numpy as jnp
from jax.experimental import pallas as pl


def kernel(x, edge_index, W1, a_src1, a_dst1, b1, W2, a_src2, a_dst2, b2):
    raise NotImplementedError("write your pallas kernel here")


# speedup vs baseline: 22.6676x; 22.6676x over previous
import jax
import jax.numpy as jnp
from jax import lax
from jax.experimental import pallas as pl
from jax.experimental.pallas import tpu as pltpu
from jax.experimental.pallas import tpu_sc as plsc

N = 10000
D = 128
HD = 64
NC, NS, L = 2, 16, 16
NW = NC * NS
C = 128
NROWS = 10240
NPAD = N + 16
ZR = NROWS // NS
BM = 400


def _proj_body(x_ref, w_ref, amat_ref, hlo_ref, hhi_ref, a_ref):
    h = jnp.dot(x_ref[...], w_ref[...], preferred_element_type=jnp.float32)
    hlo_ref[...] = h[:, 0:HD]
    hhi_ref[...] = h[:, HD:D]
    a_ref[...] = jnp.dot(h, amat_ref[...], preferred_element_type=jnp.float32)


_PROJ_OUT = [
    jax.ShapeDtypeStruct((N, HD), jnp.float32),
    jax.ShapeDtypeStruct((N, HD), jnp.float32),
    jax.ShapeDtypeStruct((N, D), jnp.float32),
]
_PROJ_OUT_SPECS = [
    pl.BlockSpec((BM, HD), lambda i: (i, 0)),
    pl.BlockSpec((BM, HD), lambda i: (i, 0)),
    pl.BlockSpec((BM, D), lambda i: (i, 0)),
]


def _proj(x, w, amat):
    k = x.shape[1]
    return pl.pallas_call(
        _proj_body,
        grid=(N // BM,),
        in_specs=[
            pl.BlockSpec((BM, k), lambda i: (i, 0)),
            pl.BlockSpec((k, D), lambda i: (0, 0)),
            pl.BlockSpec((D, D), lambda i: (0, 0)),
        ],
        out_specs=_PROJ_OUT_SPECS,
        out_shape=_PROJ_OUT,
    )(x, w, amat)


def _comb_proj_body(num_ref, den_ref, b_ref, w_ref, amat_ref,
                    hlo_ref, hhi_ref, a_ref):
    n = jnp.concatenate(
        [num_ref[0, 0] + num_ref[1, 0], num_ref[0, 1] + num_ref[1, 1]], axis=1)
    d = den_ref[0] + den_ref[1]
    lane = lax.broadcasted_iota(jnp.int32, (BM, D), 1)
    dexp = jnp.where(lane < HD, d[:, 0:1], d[:, 1:2])
    v = n / dexp + b_ref[...]
    v = jnp.where(v > 0, v, jnp.exp(v) - 1.0)
    h = jnp.dot(v, w_ref[...], preferred_element_type=jnp.float32)
    hlo_ref[...] = h[:, 0:HD]
    hhi_ref[...] = h[:, HD:D]
    a_ref[...] = jnp.dot(h, amat_ref[...], preferred_element_type=jnp.float32)


def _comb_proj(num, den, b, w, amat):
    return pl.pallas_call(
        _comb_proj_body,
        grid=(N // BM,),
        in_specs=[
            pl.BlockSpec((NC, 2, BM, HD), lambda i: (0, 0, i, 0)),
            pl.BlockSpec((NC, BM, L), lambda i: (0, i, 0)),
            pl.BlockSpec((1, D), lambda i: (0, 0)),
            pl.BlockSpec((D, D), lambda i: (0, 0)),
            pl.BlockSpec((D, D), lambda i: (0, 0)),
        ],
        out_specs=_PROJ_OUT_SPECS,
        out_shape=_PROJ_OUT,
    )(num, den, b, w, amat)


def _final_body(num_ref, den_ref, b_ref, out_ref):
    n = jnp.concatenate(
        [num_ref[0, 0] + num_ref[1, 0], num_ref[0, 1] + num_ref[1, 1]], axis=1)
    d = den_ref[0][:, 0:1] + den_ref[1][:, 0:1]
    out_ref[...] = n / d + b_ref[...]


def _final(num, den, b):
    return pl.pallas_call(
        _final_body,
        grid=(N // BM,),
        in_specs=[
            pl.BlockSpec((NC, 2, BM, HD), lambda i: (0, 0, i, 0)),
            pl.BlockSpec((NC, BM, L), lambda i: (0, i, 0)),
            pl.BlockSpec((1, D), lambda i: (0, 0)),
        ],
        out_specs=pl.BlockSpec((BM, D), lambda i: (i, 0)),
        out_shape=jax.ShapeDtypeStruct((N, D), jnp.float32),
    )(num, den, b)


def _make_edge_fn(heads, nch):
    mesh = plsc.VectorSubcoreMesh(core_axis_name="c", subcore_axis_name="s")
    npadh = NPAD * heads

    def body(hlo_hbm, hhi_hbm, as_hbm, ad_hbm, src_hbm, dst_hbm,
             num_hbm, den_hbm,
             as_v, ad_v, src_v, dst_v, srcc, dstc, gbuf, sbuf, wrow, wbuf,
             num_sh, den_sh, gsem):
        cid = lax.axis_index("c")
        sid = lax.axis_index("s")
        wid = sid * NC + cid

        pltpu.sync_copy(as_hbm, as_v)
        pltpu.sync_copy(ad_hbm, ad_v)
        pltpu.sync_copy(src_hbm.at[wid], src_v)
        pltpu.sync_copy(dst_hbm.at[wid], dst_v)

        zf = jnp.zeros((L,), jnp.float32)
        iot = lax.iota(jnp.int32, L)
        zbase = sid * ZR

        for hp in range(2):
            h_hbm = hlo_hbm if hp == 0 else hhi_hbm
            wh = hp if heads == 2 else 0

            def zrow(i, _):
                for v in range(HD // L):
                    sbuf[i, pl.ds(v * L, L)] = zf
                if hp == 0:
                    wrow[i, pl.ds(0, L)] = zf
                return 0

            lax.fori_loop(0, C, zrow, 0)
            for k in range(ZR // C):
                pltpu.sync_copy(sbuf, num_sh.at[pl.ds(zbase + k * C, C)])
                if hp == 0:
                    pltpu.sync_copy(wrow, den_sh.at[pl.ds(zbase + k * C, C)])
            plsc.subcore_barrier()

            def chunk(j, _):
                for g in range(C // L):
                    s16 = src_v[j, pl.ds(g * L, L)]
                    d16 = dst_v[j, pl.ds(g * L, L)]
                    srcc[pl.ds(g * L, L)] = s16
                    dstc[pl.ds(g * L, L)] = d16
                    rows16 = iot + g * L
                    for hh in range(heads):
                        av = plsc.load_gather(as_v, [s16 * heads + hh])
                        bv = plsc.load_gather(ad_v, [d16 * heads + hh])
                        e = av + bv
                        e = jnp.maximum(e, 0.2 * e)
                        w = jnp.exp(e)
                        if hh == wh:
                            wbuf[pl.ds(g * L, L)] = w
                        if hp == 0:
                            plsc.store_scatter(
                                wrow, [rows16, jnp.full((L,), hh, jnp.int32)], w)

                pltpu.async_copy(h_hbm.at[srcc], gbuf, gsem).wait()

                def edge(i, _):
                    w0 = plsc.load_gather(wbuf, [jnp.full((L,), i, jnp.int32)])
                    for v in range(HD // L):
                        sbuf[i, pl.ds(v * L, L)] = gbuf[i, pl.ds(v * L, L)] * w0
                    return 0

                lax.fori_loop(0, C, edge, 0)
                pltpu.sync_copy(sbuf, num_sh.at[dstc], add=True)
                if hp == 0:
                    pltpu.sync_copy(wrow, den_sh.at[dstc], add=True)
                return 0

            lax.fori_loop(0, nch, chunk, 0)
            plsc.subcore_barrier()

            pltpu.sync_copy(num_sh.at[pl.ds(zbase, ZR)],
                            num_hbm.at[cid, hp, pl.ds(zbase, ZR)])
            if hp == 0:
                pltpu.sync_copy(den_sh.at[pl.ds(zbase, ZR)],
                                den_hbm.at[cid, pl.ds(zbase, ZR)])

    return pl.kernel(
        body,
        out_type=(
            jax.ShapeDtypeStruct((NC, 2, NROWS, HD), jnp.float32),
            jax.ShapeDtypeStruct((NC, NROWS, L), jnp.float32),
        ),
        mesh=mesh,
        compiler_params=pltpu.CompilerParams(needs_layout_passes=False,
                                             use_tc_tiling_on_sc=False),
        scratch_types=[
            pltpu.VMEM((npadh,), jnp.float32),
            pltpu.VMEM((npadh,), jnp.float32),
            pltpu.VMEM((nch, C), jnp.int32),
            pltpu.VMEM((nch, C), jnp.int32),
            pltpu.VMEM((C,), jnp.int32),
            pltpu.VMEM((C,), jnp.int32),
            pltpu.VMEM((C, HD), jnp.float32),
            pltpu.VMEM((C, HD), jnp.float32),
            pltpu.VMEM((C, L), jnp.float32),
            pltpu.VMEM((C,), jnp.float32),
            pltpu.VMEM_SHARED((NROWS, HD), jnp.float32),
            pltpu.VMEM_SHARED((NROWS, L), jnp.float32),
            pltpu.SemaphoreType.DMA,
        ],
    )


def _alpha_mat1(a_src, a_dst):
    z = jnp.zeros((HD,), jnp.float32)
    c0 = jnp.concatenate([a_src[0], z])
    c1 = jnp.concatenate([z, a_src[1]])
    c2 = jnp.concatenate([a_dst[0], z])
    c3 = jnp.concatenate([z, a_dst[1]])
    rest = jnp.zeros((D, D - 4), jnp.float32)
    return jnp.concatenate(
        [c0[:, None], c1[:, None], c2[:, None], c3[:, None], rest], axis=1)


def _alpha_mat2(a_src, a_dst):
    rest = jnp.zeros((D, D - 2), jnp.float32)
    return jnp.concatenate([a_src[0][:, None], a_dst[0][:, None], rest], axis=1)


def _pad_flat(a):
    return jnp.pad(a, ((0, NPAD - N), (0, 0))).reshape(-1)


def kernel(x, edge_index, W1, a_src1, a_dst1, b1, W2, a_src2, a_dst2, b2):
    e_tot = edge_index.shape[1] + N
    nch = -(-e_tot // (NW * C))
    e_pad = NW * nch * C

    loops = jnp.arange(N, dtype=jnp.int32)
    src = jnp.concatenate([
        edge_index[0].astype(jnp.int32), loops,
        jnp.zeros((e_pad - e_tot,), jnp.int32)]).reshape(NW, nch, C)
    dst = jnp.concatenate([
        edge_index[1].astype(jnp.int32), loops,
        jnp.full((e_pad - e_tot,), N, jnp.int32)]).reshape(NW, nch, C)

    edge1 = _make_edge_fn(2, nch)
    edge2 = _make_edge_fn(1, nch)

    h1lo, h1hi, aall1 = _proj(x, W1, _alpha_mat1(a_src1, a_dst1))
    num1, den1 = edge1(h1lo, h1hi, _pad_flat(aall1[:, 0:2]),
                       _pad_flat(aall1[:, 2:4]), src, dst)
    h2lo, h2hi, aall2 = _comb_proj(num1, den1, b1.reshape(1, D), W2,
                                   _alpha_mat2(a_src2, a_dst2))
    num2, den2 = edge2(h2lo, h2hi, _pad_flat(aall2[:, 0:1]),
                       _pad_flat(aall2[:, 1:2]), src, dst)
    return _final(num2, den2, b2.reshape(1, D))
